# Initial kernel scaffold; baseline (speedup 1.0000x reference)
import jax
import jax.numpy as jnp
from jax.experimental import pallas as pl


def kernel(x, edge_index, batch, Wl1, bl1, Wr1, g1, be1, Wl2, bl2, Wr2, g2, be2, Wl3, bl3, Wr3, g3, be3, linW, linb):
    raise NotImplementedError("write your pallas kernel here")


# speedup vs baseline: 2.4452x; 2.4452x over previous
import functools

import jax
import jax.numpy as jnp
from jax import lax
from jax.experimental import pallas as pl
from jax.experimental.pallas import tpu as pltpu
from jax.experimental.pallas import tpu_sc as plsc

N = 10000
D = 256
H = 256
C = 2
G = 64
E = 160000

NC = 2
NS = 16
HW = H // NC
K = 128
EPT = 10240
EPAD = EPT * NS
NCHUNK = EPT // K
RPT = 632
NPAD = RPT * NS
DW = 16


_TC_PARAMS = pltpu.CompilerParams(vmem_limit_bytes=100 * 1024 * 1024)


def _sc_mesh():
    return plsc.VectorSubcoreMesh(core_axis_name="c", subcore_axis_name="s")


def _sc_agg(hlflat, srcs, dstp):
    zin = jnp.zeros((NPAD, HW), jnp.float32)

    @functools.partial(
        pl.kernel,
        out_type=jax.ShapeDtypeStruct((NC * NPAD, HW), jnp.float32),
        mesh=_sc_mesh(),
        scratch_types=[
            pltpu.VMEM_SHARED((NPAD, HW), jnp.float32),
            pltpu.VMEM((K,), jnp.int32),
            pltpu.VMEM((K,), jnp.int32),
            pltpu.VMEM((K, HW), jnp.float32),
            pltpu.SemaphoreType.DMA,
        ],
    )
    def k(hl_hbm, srcs_hbm, dst_hbm, zin_hbm, agg_hbm, agg_sh, sidx, didx, rows, sem):
        c = lax.axis_index("c")
        s = lax.axis_index("s")
        r0 = s * RPT
        pltpu.sync_copy(zin_hbm.at[pl.ds(r0, RPT)], agg_sh.at[pl.ds(r0, RPT)])
        plsc.subcore_barrier()
        e0 = c * EPAD + s * EPT
        d0 = s * EPT

        def chunk(j, carry):
            pltpu.sync_copy(srcs_hbm.at[pl.ds(e0 + j * K, K)], sidx)
            pltpu.sync_copy(dst_hbm.at[pl.ds(d0 + j * K, K)], didx)
            pltpu.async_copy(hl_hbm.at[sidx], rows, sem).wait()
            pltpu.sync_copy(rows, agg_sh.at[didx], add=True)
            return carry

        lax.fori_loop(0, NCHUNK, chunk, 0)
        plsc.subcore_barrier()
        pltpu.sync_copy(agg_sh.at[pl.ds(r0, RPT)],
                        agg_hbm.at[pl.ds(c * NPAD + r0, RPT)])

    return k(hlflat, srcs, dstp, zin)


def _sc_deg(dstp):
    zin = jnp.zeros((NPAD, HW), jnp.float32)
    ones = jnp.ones((K, HW), jnp.float32)
    ept2 = EPT // 2
    nch2 = NCHUNK // 2

    @functools.partial(
        pl.kernel,
        out_type=jax.ShapeDtypeStruct((NC * NPAD, HW), jnp.float32),
        mesh=_sc_mesh(),
        scratch_types=[
            pltpu.VMEM_SHARED((NPAD, HW), jnp.float32),
            pltpu.VMEM((K,), jnp.int32),
            pltpu.VMEM((K, HW), jnp.float32),
        ],
    )
    def k(dst_hbm, zin_hbm, ones_hbm, deg_hbm, deg_sh, didx, ones_v):
        c = lax.axis_index("c")
        s = lax.axis_index("s")
        r0 = s * RPT
        pltpu.sync_copy(zin_hbm.at[pl.ds(r0, RPT)], deg_sh.at[pl.ds(r0, RPT)])
        pltpu.sync_copy(ones_hbm, ones_v)
        plsc.subcore_barrier()
        d0 = c * (EPAD // 2) + s * ept2

        def chunk(j, carry):
            pltpu.sync_copy(dst_hbm.at[pl.ds(d0 + j * K, K)], didx)
            pltpu.sync_copy(ones_v, deg_sh.at[didx], add=True)
            return carry

        lax.fori_loop(0, nch2, chunk, 0)
        plsc.subcore_barrier()
        pltpu.sync_copy(deg_sh.at[pl.ds(r0, RPT)],
                        deg_hbm.at[pl.ds(c * NPAD + r0, RPT)])

    return k(dstp, zin, ones)


def _tc_front(x, Wl, Wr):
    def body(x_ref, wl_ref, wr_ref, hl_ref, hr_ref):
        xv = x_ref[...]
        hl = jnp.dot(xv, wl_ref[...], preferred_element_type=jnp.float32)
        hr_ref[...] = jnp.dot(xv, wr_ref[...], preferred_element_type=jnp.float32)
        hl_ref[0:N, :] = hl[:, 0:HW]
        hl_ref[N:2 * N, :] = hl[:, HW:H]

    return pl.pallas_call(
        body,
        out_shape=[
            jax.ShapeDtypeStruct((NC * N, HW), jnp.float32),
            jax.ShapeDtypeStruct((N, H), jnp.float32),
        ],
        compiler_params=_TC_PARAMS,
    )(x, Wl, Wr)


def _combine(agg_ref, deg_ref, hr_ref, bl_ref, g_ref, be_ref):
    a0 = agg_ref[0:N, :]
    a1 = agg_ref[NPAD:NPAD + N, :]
    aggc = jnp.concatenate([a0, a1], axis=1)
    degv = deg_ref[0:N, 0:1] + deg_ref[NPAD:NPAD + N, 0:1]
    degv = jnp.maximum(degv, 1.0)
    t = aggc / degv + bl_ref[...][None, :] + hr_ref[...]
    m = jnp.mean(t, axis=0, keepdims=True)
    v = jnp.mean((t - m) ** 2, axis=0, keepdims=True)
    h = (t - m) * lax.rsqrt(v + 1e-5) * g_ref[...][None, :] + be_ref[...][None, :]
    return jnp.maximum(h, 0.0)


def _tc_mid(agg, deg, hr, bl, g, be, Wl, Wr):
    def body(agg_ref, deg_ref, hr_ref, bl_ref, g_ref, be_ref, wl_ref, wr_ref,
             hl_ref, hr2_ref):
        h = _combine(agg_ref, deg_ref, hr_ref, bl_ref, g_ref, be_ref)
        hl = jnp.dot(h, wl_ref[...], preferred_element_type=jnp.float32)
        hr2_ref[...] = jnp.dot(h, wr_ref[...], preferred_element_type=jnp.float32)
        hl_ref[0:N, :] = hl[:, 0:HW]
        hl_ref[N:2 * N, :] = hl[:, HW:H]

    return pl.pallas_call(
        body,
        out_shape=[
            jax.ShapeDtypeStruct((NC * N, HW), jnp.float32),
            jax.ShapeDtypeStruct((N, H), jnp.float32),
        ],
        compiler_params=_TC_PARAMS,
    )(agg, deg, hr, bl, g, be, Wl, Wr)


def _tc_final(agg, deg, hr, bl, g, be, batch, linWp, linbp):
    def body(agg_ref, deg_ref, hr_ref, bl_ref, g_ref, be_ref, b_ref, w_ref,
             wb_ref, out_ref):
        h = _combine(agg_ref, deg_ref, hr_ref, bl_ref, g_ref, be_ref)
        bb = b_ref[...]
        gids = lax.broadcasted_iota(jnp.int32, (G, N), 0)
        oh = (bb[None, :] == gids).astype(jnp.float32)
        psum = jnp.dot(oh, h, preferred_element_type=jnp.float32)
        cnt = jnp.sum(oh, axis=1, keepdims=True)
        pooled = psum / jnp.maximum(cnt, 1.0)
        out_ref[...] = (
            jnp.dot(pooled, w_ref[...], preferred_element_type=jnp.float32)
            + wb_ref[...][None, :]
        )

    return pl.pallas_call(
        body,
        out_shape=jax.ShapeDtypeStruct((G, 128), jnp.float32),
        compiler_params=_TC_PARAMS,
    )(agg, deg, hr, bl, g, be, batch, linWp, linbp)


def kernel(x, edge_index, batch, Wl1, bl1, Wr1, g1, be1, Wl2, bl2, Wr2, g2,
           be2, Wl3, bl3, Wr3, g3, be3, linW, linb):
    src = edge_index[0]
    dst = edge_index[1]
    src_p = jnp.concatenate([src, jnp.zeros((EPAD - E,), jnp.int32)])
    dst_p = jnp.concatenate([dst, jnp.full((EPAD - E,), N, jnp.int32)])
    srcs = jnp.concatenate([src_p, src_p + N])

    linWp = jnp.zeros((H, 128), jnp.float32).at[:, :C].set(linW)
    linbp = jnp.zeros((128,), jnp.float32).at[:C].set(linb)

    deg = _sc_deg(dst_p)
    hl1, hr1 = _tc_front(x, Wl1, Wr1)
    agg1 = _sc_agg(hl1, srcs, dst_p)
    hl2, hr2 = _tc_mid(agg1, deg, hr1, bl1, g1, be1, Wl2, Wr2)
    agg2 = _sc_agg(hl2, srcs, dst_p)
    hl3, hr3 = _tc_mid(agg2, deg, hr2, bl2, g2, be2, Wl3, Wr3)
    agg3 = _sc_agg(hl3, srcs, dst_p)
    outp = _tc_final(agg3, deg, hr3, bl3, g3, be3, batch, linWp, linbp)
    return outp[:, :C]


# speedup vs baseline: 3.2879x; 1.3447x over previous
import functools

import jax
import jax.numpy as jnp
from jax import lax
from jax.experimental import pallas as pl
from jax.experimental.pallas import tpu as pltpu
from jax.experimental.pallas import tpu_sc as plsc

N = 10000
D = 256
H = 256
C = 2
G = 64
E = 160000

NC = 2
NS = 16
HW = H // NC
K = 128
EPT = 10240
EPAD = EPT * NS
NCHUNK = EPT // K
RPT = 632
NPAD = RPT * NS
DW = 16


_TC_PARAMS = pltpu.CompilerParams(vmem_limit_bytes=100 * 1024 * 1024)


def _sc_mesh():
    return plsc.VectorSubcoreMesh(core_axis_name="c", subcore_axis_name="s")


def _sc_agg(hlflat, srcs2, dst3):
    zin = jnp.zeros((NPAD, HW), jnp.float32)

    @functools.partial(
        pl.kernel,
        out_type=jax.ShapeDtypeStruct((NC * NPAD, HW), jnp.float32),
        mesh=_sc_mesh(),
        scratch_types=[
            pltpu.VMEM_SHARED((NPAD, HW), jnp.float32),
            pltpu.VMEM((NCHUNK, K), jnp.int32),
            pltpu.VMEM((K,), jnp.int32),
            pltpu.VMEM((K,), jnp.int32),
            pltpu.VMEM((K, HW), jnp.float32),
            pltpu.VMEM((K, HW), jnp.float32),
            pltpu.SemaphoreType.DMA,
            pltpu.SemaphoreType.DMA,
        ],
    )
    def k(hl_hbm, srcs_hbm, dst_hbm, zin_hbm, agg_hbm,
          agg_sh, didx, sidx0, sidx1, rows0, rows1, sem0, sem1):
        c = lax.axis_index("c")
        s = lax.axis_index("s")
        r0 = s * RPT
        w = c * NS + s
        row0 = w * NCHUNK
        pltpu.sync_copy(zin_hbm.at[pl.ds(r0, RPT)], agg_sh.at[pl.ds(r0, RPT)])
        pltpu.sync_copy(dst_hbm.at[s], didx)
        plsc.subcore_barrier()

        bufs = ((rows0, sidx0, sem0), (rows1, sidx1, sem1))
        pltpu.sync_copy(srcs_hbm.at[row0], sidx0)
        pltpu.async_copy(hl_hbm.at[sidx0], rows0, sem0)
        pltpu.sync_copy(srcs_hbm.at[row0 + 1], sidx1)
        pltpu.async_copy(hl_hbm.at[sidx1], rows1, sem1)

        def pair(jj, carry):
            for b, (rows, sidx, sem) in enumerate(bufs):
                j = jj * 2 + b
                pltpu.make_async_copy(hl_hbm.at[pl.ds(0, K)], rows, sem).wait()
                pltpu.sync_copy(rows, agg_sh.at[didx.at[j]], add=True)
                pltpu.sync_copy(srcs_hbm.at[row0 + j + 2], sidx)
                pltpu.async_copy(hl_hbm.at[sidx], rows, sem)
            return carry

        lax.fori_loop(0, NCHUNK // 2 - 1, pair, 0)
        for b, (rows, sidx, sem) in enumerate(bufs):
            j = NCHUNK - 2 + b
            pltpu.make_async_copy(hl_hbm.at[pl.ds(0, K)], rows, sem).wait()
            pltpu.sync_copy(rows, agg_sh.at[didx.at[j]], add=True)

        plsc.subcore_barrier()
        pltpu.sync_copy(agg_sh.at[pl.ds(r0, RPT)],
                        agg_hbm.at[pl.ds(c * NPAD + r0, RPT)])

    return k(hlflat, srcs2, dst3, zin)


def _sc_deg(dstd3):
    zin = jnp.zeros((NPAD, HW), jnp.float32)
    ones = jnp.ones((K, HW), jnp.float32)
    nch2 = NCHUNK // 2

    @functools.partial(
        pl.kernel,
        out_type=jax.ShapeDtypeStruct((NC * NPAD, HW), jnp.float32),
        mesh=_sc_mesh(),
        scratch_types=[
            pltpu.VMEM_SHARED((NPAD, HW), jnp.float32),
            pltpu.VMEM((nch2, K), jnp.int32),
            pltpu.VMEM((K, HW), jnp.float32),
        ],
    )
    def k(dst_hbm, zin_hbm, ones_hbm, deg_hbm, deg_sh, didx, ones_v):
        c = lax.axis_index("c")
        s = lax.axis_index("s")
        r0 = s * RPT
        w = c * NS + s
        pltpu.sync_copy(zin_hbm.at[pl.ds(r0, RPT)], deg_sh.at[pl.ds(r0, RPT)])
        pltpu.sync_copy(ones_hbm, ones_v)
        pltpu.sync_copy(dst_hbm.at[w], didx)
        plsc.subcore_barrier()

        def chunk(j, carry):
            pltpu.sync_copy(ones_v, deg_sh.at[didx.at[j]], add=True)
            return carry

        lax.fori_loop(0, nch2, chunk, 0)
        plsc.subcore_barrier()
        pltpu.sync_copy(deg_sh.at[pl.ds(r0, RPT)],
                        deg_hbm.at[pl.ds(c * NPAD + r0, RPT)])

    return k(dstd3, zin, ones)


def _tc_front(x, Wl, Wr):
    def body(x_ref, wl_ref, wr_ref, hl_ref, hr_ref):
        xv = x_ref[...]
        hl = jnp.dot(xv, wl_ref[...], preferred_element_type=jnp.float32)
        hr_ref[...] = jnp.dot(xv, wr_ref[...], preferred_element_type=jnp.float32)
        hl_ref[0:N, :] = hl[:, 0:HW]
        hl_ref[N:2 * N, :] = hl[:, HW:H]

    return pl.pallas_call(
        body,
        out_shape=[
            jax.ShapeDtypeStruct((NC * N, HW), jnp.float32),
            jax.ShapeDtypeStruct((N, H), jnp.float32),
        ],
        compiler_params=_TC_PARAMS,
    )(x, Wl, Wr)


def _combine(agg_ref, deg_ref, hr_ref, bl_ref, g_ref, be_ref):
    a0 = agg_ref[0:N, :]
    a1 = agg_ref[NPAD:NPAD + N, :]
    aggc = jnp.concatenate([a0, a1], axis=1)
    degv = deg_ref[0:N, 0:1] + deg_ref[NPAD:NPAD + N, 0:1]
    degv = jnp.maximum(degv, 1.0)
    t = aggc / degv + bl_ref[...][None, :] + hr_ref[...]
    m = jnp.mean(t, axis=0, keepdims=True)
    v = jnp.mean((t - m) ** 2, axis=0, keepdims=True)
    h = (t - m) * lax.rsqrt(v + 1e-5) * g_ref[...][None, :] + be_ref[...][None, :]
    return jnp.maximum(h, 0.0)


def _tc_mid(agg, deg, hr, bl, g, be, Wl, Wr):
    def body(agg_ref, deg_ref, hr_ref, bl_ref, g_ref, be_ref, wl_ref, wr_ref,
             hl_ref, hr2_ref):
        h = _combine(agg_ref, deg_ref, hr_ref, bl_ref, g_ref, be_ref)
        hl = jnp.dot(h, wl_ref[...], preferred_element_type=jnp.float32)
        hr2_ref[...] = jnp.dot(h, wr_ref[...], preferred_element_type=jnp.float32)
        hl_ref[0:N, :] = hl[:, 0:HW]
        hl_ref[N:2 * N, :] = hl[:, HW:H]

    return pl.pallas_call(
        body,
        out_shape=[
            jax.ShapeDtypeStruct((NC * N, HW), jnp.float32),
            jax.ShapeDtypeStruct((N, H), jnp.float32),
        ],
        compiler_params=_TC_PARAMS,
    )(agg, deg, hr, bl, g, be, Wl, Wr)


def _tc_final(agg, deg, hr, bl, g, be, batch, linWp, linbp):
    def body(agg_ref, deg_ref, hr_ref, bl_ref, g_ref, be_ref, b_ref, w_ref,
             wb_ref, out_ref):
        h = _combine(agg_ref, deg_ref, hr_ref, bl_ref, g_ref, be_ref)
        bb = b_ref[...]
        gids = lax.broadcasted_iota(jnp.int32, (G, N), 0)
        oh = (bb[None, :] == gids).astype(jnp.float32)
        psum = jnp.dot(oh, h, preferred_element_type=jnp.float32)
        cnt = jnp.sum(oh, axis=1, keepdims=True)
        pooled = psum / jnp.maximum(cnt, 1.0)
        out_ref[...] = (
            jnp.dot(pooled, w_ref[...], preferred_element_type=jnp.float32)
            + wb_ref[...][None, :]
        )

    return pl.pallas_call(
        body,
        out_shape=jax.ShapeDtypeStruct((G, 128), jnp.float32),
        compiler_params=_TC_PARAMS,
    )(agg, deg, hr, bl, g, be, batch, linWp, linbp)


def kernel(x, edge_index, batch, Wl1, bl1, Wr1, g1, be1, Wl2, bl2, Wr2, g2,
           be2, Wl3, bl3, Wr3, g3, be3, linW, linb):
    src = edge_index[0]
    dst = edge_index[1]
    src_p = jnp.concatenate([src, jnp.zeros((EPAD - E,), jnp.int32)])
    dst_p = jnp.concatenate([dst, jnp.full((EPAD - E,), N, jnp.int32)])
    srcs2 = jnp.concatenate([src_p, src_p + N]).reshape(NC * NS * NCHUNK, K)
    dst3 = dst_p.reshape(NS, NCHUNK, K)
    dstd3 = dst_p.reshape(NC * NS, NCHUNK // 2, K)

    linWp = jnp.zeros((H, 128), jnp.float32).at[:, :C].set(linW)
    linbp = jnp.zeros((128,), jnp.float32).at[:C].set(linb)

    deg = _sc_deg(dstd3)
    hl1, hr1 = _tc_front(x, Wl1, Wr1)
    agg1 = _sc_agg(hl1, srcs2, dst3)
    hl2, hr2 = _tc_mid(agg1, deg, hr1, bl1, g1, be1, Wl2, Wr2)
    agg2 = _sc_agg(hl2, srcs2, dst3)
    hl3, hr3 = _tc_mid(agg2, deg, hr2, bl2, g2, be2, Wl3, Wr3)
    agg3 = _sc_agg(hl3, srcs2, dst3)
    outp = _tc_final(agg3, deg, hr3, bl3, g3, be3, batch, linWp, linbp)
    return outp[:, :C]
